# baseline (speedup 1.0000x reference)
import functools

import jax
import jax.numpy as jnp
from jax import lax
from jax.experimental import pallas as pl
from jax.experimental.pallas import tpu as pltpu
from jax.experimental.pallas import tpu_sc as plsc

N = 10000
E = 320000
D = 128

NC = 2
NS = 16
NW = NC * NS
CHUNK = 128
SUP = 2
EPW = E // NW
NSUP = 40
SPB = 4
NBLK = NSUP // SPB
EPW_PAD = NSUP * SUP * CHUNK

ROWS_PAD = 10240
OROWS = ROWS_PAD // NS

BLK = 1000


def _sc_agg_body(x_hbm, src_hbm, dst_hbm, zero_hbm, out_hbm,
                 srcblk0, srcblk1, dstblk0, dstblk1, rows_v, agg_s,
                 gsem, isem0, isem1):
    c = lax.axis_index("c")
    s = lax.axis_index("s")
    wid = c * NS + s
    srcblk = (srcblk0, srcblk1)
    dstblk = (dstblk0, dstblk1)
    isem = (isem0, isem1)

    pltpu.sync_copy(zero_hbm.at[pl.ds(s * OROWS, OROWS)],
                    agg_s.at[pl.ds(s * OROWS, OROWS)])

    pltpu.async_copy(src_hbm.at[wid, 0], srcblk[0], isem[0])
    pltpu.async_copy(dst_hbm.at[wid, 0], dstblk[0], isem[0])
    pltpu.async_copy(src_hbm.at[wid, 1], srcblk[1], isem[1])
    pltpu.async_copy(dst_hbm.at[wid, 1], dstblk[1], isem[1])

    plsc.subcore_barrier()

    def _block_pair(ii, carry):
        for kb in range(2):
            b = 2 * ii + kb
            for k in range(SPB):
                if k == 0:
                    pltpu.make_async_copy(src_hbm.at[wid, b],
                                          srcblk[kb], isem[kb]).wait()
                    pltpu.make_async_copy(dst_hbm.at[wid, b],
                                          dstblk[kb], isem[kb]).wait()
                sl = pl.ds(k * SUP * CHUNK, SUP * CHUNK)
                pltpu.async_copy(x_hbm.at[srcblk[kb].at[sl]], rows_v,
                                 gsem).wait()
                pltpu.sync_copy(rows_v, agg_s.at[dstblk[kb].at[sl]],
                                add=True)
                if k == SPB - 1:
                    @pl.when(b + 2 < NBLK)
                    def _():
                        pltpu.async_copy(src_hbm.at[wid, b + 2],
                                         srcblk[kb], isem[kb])
                        pltpu.async_copy(dst_hbm.at[wid, b + 2],
                                         dstblk[kb], isem[kb])
        return carry
    lax.fori_loop(0, NBLK // 2, _block_pair, 0)

    plsc.subcore_barrier()

    pltpu.sync_copy(agg_s.at[pl.ds(s * OROWS, OROWS)],
                    out_hbm.at[c, pl.ds(s * OROWS, OROWS)])


@functools.partial(
    pl.kernel,
    out_type=jax.ShapeDtypeStruct((NC, ROWS_PAD, D), jnp.float32),
    mesh=plsc.VectorSubcoreMesh(core_axis_name="c", subcore_axis_name="s"),
    scratch_types=[
        pltpu.VMEM((SPB * SUP * CHUNK,), jnp.int32),
        pltpu.VMEM((SPB * SUP * CHUNK,), jnp.int32),
        pltpu.VMEM((SPB * SUP * CHUNK,), jnp.int32),
        pltpu.VMEM((SPB * SUP * CHUNK,), jnp.int32),
        pltpu.VMEM((SUP * CHUNK, D), jnp.float32),
        pltpu.VMEM_SHARED((ROWS_PAD, D), jnp.float32),
        pltpu.SemaphoreType.DMA,
        pltpu.SemaphoreType.DMA,
        pltpu.SemaphoreType.DMA,
    ],
)
def _sc_agg(x_hbm, src_hbm, dst_hbm, zero_hbm, out_hbm,
            srcblk0, srcblk1, dstblk0, dstblk1, rows_v, agg_s,
            gsem, isem0, isem1):
    _sc_agg_body(x_hbm, src_hbm, dst_hbm, zero_hbm, out_hbm,
                 srcblk0, srcblk1, dstblk0, dstblk1, rows_v, agg_s,
                 gsem, isem0, isem1)


def _elu(h):
    return jnp.where(h > 0, h, jnp.exp(h) - 1.0)


def _mlp_body(x_ref, a_ref, w1_ref, b1_ref, w2_ref, b2_ref, o_ref):
    h = x_ref[...] + a_ref[0] + a_ref[1]
    h = jnp.dot(h, w1_ref[...], preferred_element_type=jnp.float32) + b1_ref[...]
    h = _elu(h)
    h = jnp.dot(h, w2_ref[...], preferred_element_type=jnp.float32) + b2_ref[...]
    o_ref[...] = _elu(h)


def _final_body(x_ref, a_ref, w1_ref, b1_ref, w2_ref, b2_ref,
                l1w_ref, l1b_ref, l2w_ref, l2b_ref, o_ref):
    h = x_ref[...] + a_ref[0] + a_ref[1]
    h = jnp.dot(h, w1_ref[...], preferred_element_type=jnp.float32) + b1_ref[...]
    h = _elu(h)
    h = jnp.dot(h, w2_ref[...], preferred_element_type=jnp.float32) + b2_ref[...]
    h = _elu(h)
    h = jnp.dot(h, l1w_ref[...], preferred_element_type=jnp.float32) + l1b_ref[...]
    h = _elu(h)
    o_ref[...] = jnp.dot(h, l2w_ref[...], preferred_element_type=jnp.float32) + l2b_ref[...]


def _row_specs(n_weights):
    x_spec = pl.BlockSpec((BLK, D), lambda i: (i, 0))
    a_spec = pl.BlockSpec((NC, BLK, D), lambda i: (0, i, 0))
    w_specs = [pl.BlockSpec(memory_space=pltpu.VMEM) for _ in range(n_weights)]
    return [x_spec, a_spec] + w_specs


def _mlp(x, agg, w1, b1, w2, b2):
    return pl.pallas_call(
        _mlp_body,
        grid=(N // BLK,),
        in_specs=_row_specs(4),
        out_specs=pl.BlockSpec((BLK, D), lambda i: (i, 0)),
        out_shape=jax.ShapeDtypeStruct((N, D), jnp.float32),
    )(x, agg, w1, b1, w2, b2)


def _final(x, agg, w1, b1, w2, b2, l1w, l1b, l2w, l2b):
    return pl.pallas_call(
        _final_body,
        grid=(N // BLK,),
        in_specs=_row_specs(8),
        out_specs=pl.BlockSpec((BLK, D), lambda i: (i, 0)),
        out_shape=jax.ShapeDtypeStruct((N, D), jnp.float32),
    )(x, agg, w1, b1, w2, b2, l1w, l1b, l2w, l2b)


def _fold_bn(w1, b1, bnw, bnb):
    scale = bnw / jnp.sqrt(jnp.float32(1.0 + 1e-5))
    return w1 * scale[None, :], b1 * scale + bnb


def kernel(x, edge_index, c1_w1, c1_b1, c1_bnw, c1_bnb, c1_w2, c1_b2,
           c2_w1, c2_b1, c2_bnw, c2_bnb, c2_w2, c2_b2,
           c3_w1, c3_b1, c3_bnw, c3_bnb, c3_w2, c3_b2,
           l1_w, l1_b, l2_w, l2_b):
    pad = EPW_PAD - EPW
    src = edge_index[0].reshape(NW, EPW)
    dst = edge_index[1].reshape(NW, EPW)
    src_p = jnp.pad(src, ((0, 0), (0, pad))).reshape(
        NW, NBLK, SPB * SUP * CHUNK)
    dst_p = jnp.pad(dst, ((0, 0), (0, pad)), constant_values=N).reshape(
        NW, NBLK, SPB * SUP * CHUNK)
    zero = jnp.zeros((ROWS_PAD, D), jnp.float32)

    w1a, b1a = _fold_bn(c1_w1, c1_b1, c1_bnw, c1_bnb)
    w1b, b1b = _fold_bn(c2_w1, c2_b1, c2_bnw, c2_bnb)
    w1c, b1c = _fold_bn(c3_w1, c3_b1, c3_bnw, c3_bnb)

    agg1 = _sc_agg(x, src_p, dst_p, zero)
    h1 = _mlp(x, agg1, w1a, b1a, c1_w2, c1_b2)
    agg2 = _sc_agg(h1, src_p, dst_p, zero)
    h2 = _mlp(h1, agg2, w1b, b1b, c2_w2, c2_b2)
    agg3 = _sc_agg(h2, src_p, dst_p, zero)
    return _final(h2, agg3, w1c, b1c, c3_w2, c3_b2, l1_w, l1_b, l2_w, l2_b)


# speedup vs baseline: 1.7079x; 1.7079x over previous
import functools

import jax
import jax.numpy as jnp
from jax import lax
from jax.experimental import pallas as pl
from jax.experimental.pallas import tpu as pltpu
from jax.experimental.pallas import tpu_sc as plsc

N = 10000
E = 320000
D = 128

NC = 2
NS = 16
NW = NC * NS
SUPW = 224
EPW = E // NW
NSUP = 45
EPW_PAD = NSUP * SUPW

ROWS_PAD = 10240
OROWS = ROWS_PAD // NS

BLK = 1000


def _sc_agg_body(x_hbm, src_hbm, dst_hbm, zero_hbm, out_hbm,
                 src_v, dst_v, rows_v, agg_s, gsem):
    c = lax.axis_index("c")
    s = lax.axis_index("s")
    wid = c * NS + s

    pltpu.sync_copy(zero_hbm.at[pl.ds(s * OROWS, OROWS)],
                    agg_s.at[pl.ds(s * OROWS, OROWS)])

    pltpu.sync_copy(src_hbm.at[wid], src_v)
    pltpu.sync_copy(dst_hbm.at[wid], dst_v)

    plsc.subcore_barrier()

    def _edge_step(j, carry):
        sl = pl.ds(j * SUPW, SUPW)
        pltpu.async_copy(x_hbm.at[src_v.at[sl]], rows_v, gsem).wait()
        pltpu.sync_copy(rows_v, agg_s.at[dst_v.at[sl]], add=True)
        return carry
    lax.fori_loop(0, NSUP, _edge_step, 0)

    plsc.subcore_barrier()

    pltpu.sync_copy(agg_s.at[pl.ds(s * OROWS, OROWS)],
                    out_hbm.at[c, pl.ds(s * OROWS, OROWS)])


@functools.partial(
    pl.kernel,
    out_type=jax.ShapeDtypeStruct((NC, ROWS_PAD, D), jnp.float32),
    mesh=plsc.VectorSubcoreMesh(core_axis_name="c", subcore_axis_name="s"),
    scratch_types=[
        pltpu.VMEM((EPW_PAD,), jnp.int32),
        pltpu.VMEM((EPW_PAD,), jnp.int32),
        pltpu.VMEM((SUPW, D), jnp.float32),
        pltpu.VMEM_SHARED((ROWS_PAD, D), jnp.float32),
        pltpu.SemaphoreType.DMA,
    ],
)
def _sc_agg(x_hbm, src_hbm, dst_hbm, zero_hbm, out_hbm,
            src_v, dst_v, rows_v, agg_s, gsem):
    _sc_agg_body(x_hbm, src_hbm, dst_hbm, zero_hbm, out_hbm,
                 src_v, dst_v, rows_v, agg_s, gsem)


def _elu(h):
    return jnp.where(h > 0, h, jnp.exp(h) - 1.0)


def _mlp_body(x_ref, a_ref, w1_ref, b1_ref, w2_ref, b2_ref, o_ref):
    h = x_ref[...] + a_ref[0] + a_ref[1]
    h = jnp.dot(h, w1_ref[...], preferred_element_type=jnp.float32) + b1_ref[...]
    h = _elu(h)
    h = jnp.dot(h, w2_ref[...], preferred_element_type=jnp.float32) + b2_ref[...]
    o_ref[...] = _elu(h)


def _final_body(x_ref, a_ref, w1_ref, b1_ref, w2_ref, b2_ref,
                l1w_ref, l1b_ref, l2w_ref, l2b_ref, o_ref):
    h = x_ref[...] + a_ref[0] + a_ref[1]
    h = jnp.dot(h, w1_ref[...], preferred_element_type=jnp.float32) + b1_ref[...]
    h = _elu(h)
    h = jnp.dot(h, w2_ref[...], preferred_element_type=jnp.float32) + b2_ref[...]
    h = _elu(h)
    h = jnp.dot(h, l1w_ref[...], preferred_element_type=jnp.float32) + l1b_ref[...]
    h = _elu(h)
    o_ref[...] = jnp.dot(h, l2w_ref[...], preferred_element_type=jnp.float32) + l2b_ref[...]


def _row_specs(n_weights):
    x_spec = pl.BlockSpec((BLK, D), lambda i: (i, 0))
    a_spec = pl.BlockSpec((NC, BLK, D), lambda i: (0, i, 0))
    w_specs = [pl.BlockSpec(memory_space=pltpu.VMEM) for _ in range(n_weights)]
    return [x_spec, a_spec] + w_specs


def _mlp(x, agg, w1, b1, w2, b2):
    return pl.pallas_call(
        _mlp_body,
        grid=(N // BLK,),
        in_specs=_row_specs(4),
        out_specs=pl.BlockSpec((BLK, D), lambda i: (i, 0)),
        out_shape=jax.ShapeDtypeStruct((N, D), jnp.float32),
    )(x, agg, w1, b1, w2, b2)


def _final(x, agg, w1, b1, w2, b2, l1w, l1b, l2w, l2b):
    return pl.pallas_call(
        _final_body,
        grid=(N // BLK,),
        in_specs=_row_specs(8),
        out_specs=pl.BlockSpec((BLK, D), lambda i: (i, 0)),
        out_shape=jax.ShapeDtypeStruct((N, D), jnp.float32),
    )(x, agg, w1, b1, w2, b2, l1w, l1b, l2w, l2b)


def _fold_bn(w1, b1, bnw, bnb):
    scale = bnw / jnp.sqrt(jnp.float32(1.0 + 1e-5))
    return w1 * scale[None, :], b1 * scale + bnb


def kernel(x, edge_index, c1_w1, c1_b1, c1_bnw, c1_bnb, c1_w2, c1_b2,
           c2_w1, c2_b1, c2_bnw, c2_bnb, c2_w2, c2_b2,
           c3_w1, c3_b1, c3_bnw, c3_bnb, c3_w2, c3_b2,
           l1_w, l1_b, l2_w, l2_b):
    pad = EPW_PAD - EPW
    src = edge_index[0].reshape(NW, EPW)
    dst = edge_index[1].reshape(NW, EPW)
    src_p = jnp.pad(src, ((0, 0), (0, pad)))
    dst_p = jnp.pad(dst, ((0, 0), (0, pad)), constant_values=N)
    zero = jnp.zeros((ROWS_PAD, D), jnp.float32)

    w1a, b1a = _fold_bn(c1_w1, c1_b1, c1_bnw, c1_bnb)
    w1b, b1b = _fold_bn(c2_w1, c2_b1, c2_bnw, c2_bnb)
    w1c, b1c = _fold_bn(c3_w1, c3_b1, c3_bnw, c3_bnb)

    agg1 = _sc_agg(x, src_p, dst_p, zero)
    h1 = _mlp(x, agg1, w1a, b1a, c1_w2, c1_b2)
    agg2 = _sc_agg(h1, src_p, dst_p, zero)
    h2 = _mlp(h1, agg2, w1b, b1b, c2_w2, c2_b2)
    agg3 = _sc_agg(h2, src_p, dst_p, zero)
    return _final(h2, agg3, w1c, b1c, c3_w2, c3_b2, l1_w, l1_b, l2_w, l2_b)
